# baseline (speedup 1.0000x reference)
import functools

import jax
import jax.numpy as jnp
from jax import lax
from jax.experimental import pallas as pl
from jax.experimental.pallas import tpu as pltpu
from jax.experimental.pallas import tpu_sc as plsc

_N = 10000
_E = 320000
_IN_CH = 128
_HID = 32
_LAT = 16

_NC = 2
_NS = 16
_NW = _NC * _NS
_CHUNK = 512
_K = 20
_NBUF = 4
_EPW = _K * _CHUNK
_E_PAD = _NW * _EPW
_N_PAD = 10240
_RPT = _N_PAD // _NS

_mesh = plsc.VectorSubcoreMesh(core_axis_name="c", subcore_axis_name="s")
_sc_params = pltpu.CompilerParams(needs_layout_passes=False,
                                  use_tc_tiling_on_sc=False)


@functools.partial(
    pl.kernel,
    out_type=jax.ShapeDtypeStruct((_NW, _N_PAD), jnp.float32),
    mesh=_mesh,
    scratch_types=[
        pltpu.VMEM((_EPW,), jnp.int32),
        pltpu.VMEM((_N_PAD,), jnp.float32),
    ],
    compiler_params=_sc_params,
)
def _sc_degree(dst_flat_hbm, out_hbm, dstv, deg):
    wid = lax.axis_index("c") * _NS + lax.axis_index("s")

    def _zero(i, _):
        deg[pl.ds(i * 16, 16)] = jnp.zeros((16,), jnp.float32)
        return 0

    lax.fori_loop(0, _N_PAD // 16, _zero, 0)
    pltpu.sync_copy(dst_flat_hbm.at[wid], dstv)
    ones = jnp.ones((16,), jnp.float32)

    def _acc(v, _):
        idx = dstv[pl.ds(v * 16, 16)]
        plsc.addupdate_scatter(deg, [idx], ones)
        return 0

    lax.fori_loop(0, _EPW // 16, _acc, 0)
    pltpu.sync_copy(deg, out_hbm.at[wid])


@functools.partial(
    pl.kernel,
    out_type=jax.ShapeDtypeStruct((_NC, _N_PAD, _HID), jnp.float32),
    mesh=_mesh,
    scratch_types=[
        pltpu.VMEM((_K, _CHUNK), jnp.int32),
        pltpu.VMEM((_K, _CHUNK), jnp.int32),
    ] + [pltpu.VMEM((_CHUNK, _HID), jnp.float32)] * _NBUF
      + [pltpu.SemaphoreType.DMA] * (2 * _NBUF)
      + [pltpu.VMEM_SHARED((_N_PAD, _HID), jnp.float32)],
    compiler_params=_sc_params,
)
def _sc_scatter_rows(table_hbm, src_hbm, dst_hbm, zeros_hbm, out_hbm,
                     srcv, dstv, *rest):
    rows = rest[:_NBUF]
    gsem = rest[_NBUF:2 * _NBUF]
    ssem = rest[2 * _NBUF:3 * _NBUF]
    acc = rest[3 * _NBUF]
    c = lax.axis_index("c")
    s = lax.axis_index("s")
    wid = c * _NS + s
    r0 = s * _RPT

    pltpu.sync_copy(zeros_hbm.at[pl.ds(r0, _RPT)], acc.at[pl.ds(r0, _RPT)])
    pltpu.sync_copy(src_hbm.at[wid], srcv)
    pltpu.sync_copy(dst_hbm.at[wid], dstv)
    plsc.subcore_barrier()

    for b in range(_NBUF):
        pltpu.async_copy(table_hbm.at[srcv.at[b]], rows[b], gsem[b])

    def _step(it, _):
        for b in range(_NBUF):
            j = it * _NBUF + b
            pltpu.make_async_copy(table_hbm.at[srcv.at[j]], rows[b],
                                  gsem[b]).wait()
            pltpu.async_copy(rows[b], acc.at[dstv.at[j]], ssem[b], add=True)
        for b in range(_NBUF):
            j = it * _NBUF + b
            pltpu.make_async_copy(rows[b], acc.at[dstv.at[j]], ssem[b]).wait()
            pltpu.async_copy(table_hbm.at[srcv.at[j + _NBUF]], rows[b],
                             gsem[b])
        return 0

    lax.fori_loop(0, _K // _NBUF - 1, _step, 0)
    for b in range(_NBUF):
        j = _K - _NBUF + b
        pltpu.make_async_copy(table_hbm.at[srcv.at[j]], rows[b], gsem[b]).wait()
        pltpu.async_copy(rows[b], acc.at[dstv.at[j]], ssem[b], add=True)
    for b in range(_NBUF):
        j = _K - _NBUF + b
        pltpu.make_async_copy(rows[b], acc.at[dstv.at[j]], ssem[b]).wait()

    plsc.subcore_barrier()
    pltpu.sync_copy(acc.at[pl.ds(r0, _RPT)], out_hbm.at[c, pl.ds(r0, _RPT)])


def _tc_prepare1_body(deg_parts, x, w1, table1, dinv):
    deg = 1.0 + jnp.sum(deg_parts[...], axis=0)
    di = lax.rsqrt(deg)
    h = jnp.dot(x[...], w1[...], preferred_element_type=jnp.float32)
    table1[...] = h * di[:, None]
    dinv[...] = di[:, None]


def _tc_prepare2_body(accs, table1, dinv, b1, wcat, table2):
    di = dinv[...]
    pre = (accs[0] + accs[1] + table1[...]) * di + b1[...][None, :]
    h = jnp.maximum(pre, 0.0)
    table2[...] = jnp.dot(h, wcat[...], preferred_element_type=jnp.float32) * di


def _tc_final_body(accs, table2, dinv, bcat, out):
    out[...] = (accs[0] + accs[1] + table2[...]) * dinv[...] + bcat[...][None, :]


def kernel(x, edge_index, W1, b1, W_mu, b_mu, W_lv, b_lv):
    src = edge_index[0]
    dst = edge_index[1]
    pad = _E_PAD - _E
    src_p = jnp.concatenate([src, jnp.full((pad,), _N, jnp.int32)])
    dst_p = jnp.concatenate([dst, jnp.full((pad,), _N, jnp.int32)])
    src_w = src_p.reshape(_NW, _K, _CHUNK)
    dst_w = dst_p.reshape(_NW, _K, _CHUNK)
    dst_flat = dst_p.reshape(_NW, _EPW)

    x_p = jnp.zeros((_N_PAD, _IN_CH), jnp.float32).at[:_N].set(x)
    wcat = jnp.concatenate([W_mu, W_lv], axis=1)
    bcat = jnp.concatenate([b_mu, b_lv])
    zeros_rows = jnp.zeros((_N_PAD, _HID), jnp.float32)

    deg_parts = _sc_degree(dst_flat)

    table1, dinv = pl.pallas_call(
        _tc_prepare1_body,
        out_shape=(
            jax.ShapeDtypeStruct((_N_PAD, _HID), jnp.float32),
            jax.ShapeDtypeStruct((_N_PAD, 1), jnp.float32),
        ),
    )(deg_parts, x_p, W1)

    acc1 = _sc_scatter_rows(table1, src_w, dst_w, zeros_rows)

    table2 = pl.pallas_call(
        _tc_prepare2_body,
        out_shape=jax.ShapeDtypeStruct((_N_PAD, _HID), jnp.float32),
    )(acc1, table1, dinv, b1, wcat)

    acc2 = _sc_scatter_rows(table2, src_w, dst_w, zeros_rows)

    out = pl.pallas_call(
        _tc_final_body,
        out_shape=jax.ShapeDtypeStruct((_N_PAD, _HID), jnp.float32),
    )(acc2, table2, dinv, bcat)

    return (out[:_N, :_LAT], out[:_N, _LAT:])


# speedup vs baseline: 1.6099x; 1.6099x over previous
import functools

import jax
import jax.numpy as jnp
from jax import lax
from jax.experimental import pallas as pl
from jax.experimental.pallas import tpu as pltpu
from jax.experimental.pallas import tpu_sc as plsc

_N = 10000
_E = 320000
_IN_CH = 128
_HID = 32
_LAT = 16

_NC = 2
_NS = 16
_NW = _NC * _NS
_CHUNK = 512
_K = 20
_NBUF = 4
_EPW = _K * _CHUNK
_E_PAD = _NW * _EPW
_N_PAD = 10240
_RPT = _N_PAD // _NS

_mesh = plsc.VectorSubcoreMesh(core_axis_name="c", subcore_axis_name="s")
_sc_params = pltpu.CompilerParams(needs_layout_passes=False,
                                  use_tc_tiling_on_sc=False)


@functools.partial(
    pl.kernel,
    out_type=jax.ShapeDtypeStruct((_NW, _N_PAD), jnp.float32),
    mesh=_mesh,
    scratch_types=[
        pltpu.VMEM((_EPW,), jnp.int32),
        pltpu.VMEM((_N_PAD,), jnp.float32),
    ],
    compiler_params=_sc_params,
)
def _sc_degree(dst_flat_hbm, out_hbm, dstv, deg):
    wid = lax.axis_index("c") * _NS + lax.axis_index("s")

    def _zero(i, _):
        deg[pl.ds(i * 16, 16)] = jnp.zeros((16,), jnp.float32)
        return 0

    lax.fori_loop(0, _N_PAD // 16, _zero, 0)
    pltpu.sync_copy(dst_flat_hbm.at[wid], dstv)
    ones = jnp.ones((16,), jnp.float32)

    def _acc(v, _):
        idx = dstv[pl.ds(v * 16, 16)]
        plsc.addupdate_scatter(deg, [idx], ones)
        return 0

    lax.fori_loop(0, _EPW // 16, _acc, 0)
    pltpu.sync_copy(deg, out_hbm.at[wid])


@functools.partial(
    pl.kernel,
    out_type=jax.ShapeDtypeStruct((_NC, _N_PAD, _HID), jnp.float32),
    mesh=_mesh,
    scratch_types=[
        pltpu.VMEM((_K, _CHUNK), jnp.int32),
        pltpu.VMEM((_K, _CHUNK), jnp.int32),
    ] + [pltpu.VMEM((_CHUNK, _HID), jnp.float32)] * _NBUF
      + [pltpu.SemaphoreType.DMA] * (2 * _NBUF)
      + [pltpu.VMEM_SHARED((_N_PAD, _HID), jnp.float32)] * 2,
    compiler_params=_sc_params,
)
def _sc_scatter_rows(table_hbm, src_hbm, dst_hbm, zeros_hbm, out_hbm,
                     srcv, dstv, *rest):
    rows = rest[:_NBUF]
    gsem = rest[_NBUF:2 * _NBUF]
    ssem = rest[2 * _NBUF:3 * _NBUF]
    acc = rest[3 * _NBUF]
    tab = rest[3 * _NBUF + 1]
    c = lax.axis_index("c")
    s = lax.axis_index("s")
    wid = c * _NS + s
    r0 = s * _RPT

    pltpu.sync_copy(zeros_hbm.at[pl.ds(r0, _RPT)], acc.at[pl.ds(r0, _RPT)])
    pltpu.sync_copy(table_hbm.at[pl.ds(r0, _RPT)], tab.at[pl.ds(r0, _RPT)])
    pltpu.sync_copy(src_hbm.at[wid], srcv)
    pltpu.sync_copy(dst_hbm.at[wid], dstv)
    plsc.subcore_barrier()

    for b in range(_NBUF):
        pltpu.async_copy(tab.at[srcv.at[b]], rows[b], gsem[b])

    def _step(it, _):
        for b in range(_NBUF):
            j = it * _NBUF + b
            pltpu.make_async_copy(tab.at[srcv.at[j]], rows[b],
                                  gsem[b]).wait()
            pltpu.async_copy(rows[b], acc.at[dstv.at[j]], ssem[b], add=True)
        for b in range(_NBUF):
            j = it * _NBUF + b
            pltpu.make_async_copy(rows[b], acc.at[dstv.at[j]], ssem[b]).wait()
            pltpu.async_copy(tab.at[srcv.at[j + _NBUF]], rows[b],
                             gsem[b])
        return 0

    lax.fori_loop(0, _K // _NBUF - 1, _step, 0)
    for b in range(_NBUF):
        j = _K - _NBUF + b
        pltpu.make_async_copy(tab.at[srcv.at[j]], rows[b], gsem[b]).wait()
        pltpu.async_copy(rows[b], acc.at[dstv.at[j]], ssem[b], add=True)
    for b in range(_NBUF):
        j = _K - _NBUF + b
        pltpu.make_async_copy(rows[b], acc.at[dstv.at[j]], ssem[b]).wait()

    plsc.subcore_barrier()
    pltpu.sync_copy(acc.at[pl.ds(r0, _RPT)], out_hbm.at[c, pl.ds(r0, _RPT)])


def _tc_prepare1_body(deg_parts, x, w1, table1, dinv):
    deg = 1.0 + jnp.sum(deg_parts[...], axis=0)
    di = lax.rsqrt(deg)
    h = jnp.dot(x[...], w1[...], preferred_element_type=jnp.float32)
    table1[...] = h * di[:, None]
    dinv[...] = di[:, None]


def _tc_prepare2_body(accs, table1, dinv, b1, wcat, table2):
    di = dinv[...]
    pre = (accs[0] + accs[1] + table1[...]) * di + b1[...][None, :]
    h = jnp.maximum(pre, 0.0)
    table2[...] = jnp.dot(h, wcat[...], preferred_element_type=jnp.float32) * di


def _tc_final_body(accs, table2, dinv, bcat, out):
    out[...] = (accs[0] + accs[1] + table2[...]) * dinv[...] + bcat[...][None, :]


def kernel(x, edge_index, W1, b1, W_mu, b_mu, W_lv, b_lv):
    src = edge_index[0]
    dst = edge_index[1]
    pad = _E_PAD - _E
    src_p = jnp.concatenate([src, jnp.full((pad,), _N, jnp.int32)])
    dst_p = jnp.concatenate([dst, jnp.full((pad,), _N, jnp.int32)])
    src_w = src_p.reshape(_NW, _K, _CHUNK)
    dst_w = dst_p.reshape(_NW, _K, _CHUNK)
    dst_flat = dst_p.reshape(_NW, _EPW)

    x_p = jnp.zeros((_N_PAD, _IN_CH), jnp.float32).at[:_N].set(x)
    wcat = jnp.concatenate([W_mu, W_lv], axis=1)
    bcat = jnp.concatenate([b_mu, b_lv])
    zeros_rows = jnp.zeros((_N_PAD, _HID), jnp.float32)

    deg_parts = _sc_degree(dst_flat)

    table1, dinv = pl.pallas_call(
        _tc_prepare1_body,
        out_shape=(
            jax.ShapeDtypeStruct((_N_PAD, _HID), jnp.float32),
            jax.ShapeDtypeStruct((_N_PAD, 1), jnp.float32),
        ),
    )(deg_parts, x_p, W1)

    acc1 = _sc_scatter_rows(table1, src_w, dst_w, zeros_rows)

    table2 = pl.pallas_call(
        _tc_prepare2_body,
        out_shape=jax.ShapeDtypeStruct((_N_PAD, _HID), jnp.float32),
    )(acc1, table1, dinv, b1, wcat)

    acc2 = _sc_scatter_rows(table2, src_w, dst_w, zeros_rows)

    out = pl.pallas_call(
        _tc_final_body,
        out_shape=jax.ShapeDtypeStruct((_N_PAD, _HID), jnp.float32),
    )(acc2, table2, dinv, bcat)

    return (out[:_N, :_LAT], out[:_N, _LAT:])


# speedup vs baseline: 1.7884x; 1.1109x over previous
import jax
import jax.numpy as jnp
from jax import lax
from jax.experimental import pallas as pl
from jax.experimental.pallas import tpu as pltpu
from jax.experimental.pallas import tpu_sc as plsc

_N = 10000
_E = 320000
_IN_CH = 128
_HID = 32
_LAT = 16

_NC = 2
_NS = 16
_NW = _NC * _NS
_EPW = _E // _NW
_CHUNK = 400
_K = _EPW // _CHUNK
_NBUF = 5
_RPT = _N // _NS

_mesh = plsc.VectorSubcoreMesh(core_axis_name="c", subcore_axis_name="s")
_sc_params = pltpu.CompilerParams(needs_layout_passes=False,
                                  use_tc_tiling_on_sc=False)


def _kernel_decorator(out_type, scratch_types):
    def deco(f):
        return pl.kernel(f, out_type=out_type, mesh=_mesh,
                         scratch_types=scratch_types,
                         compiler_params=_sc_params)
    return deco


@_kernel_decorator(
    jax.ShapeDtypeStruct((_NW, _N), jnp.float32),
    [
        pltpu.VMEM((_EPW,), jnp.int32),
        pltpu.VMEM((_N,), jnp.float32),
    ],
)
def _sc_degree(dst_flat_hbm, out_hbm, dstv, deg):
    wid = lax.axis_index("c") * _NS + lax.axis_index("s")

    def _zero(i, _):
        deg[pl.ds(i * 16, 16)] = jnp.zeros((16,), jnp.float32)
        return 0

    lax.fori_loop(0, _N // 16, _zero, 0)
    pltpu.sync_copy(dst_flat_hbm.at[wid], dstv)
    ones = jnp.ones((16,), jnp.float32)

    def _acc(v, _):
        idx = dstv[pl.ds(v * 16, 16)]
        plsc.addupdate_scatter(deg, [idx], ones)
        return 0

    lax.fori_loop(0, _EPW // 16, _acc, 0)
    pltpu.sync_copy(deg, out_hbm.at[wid])


@_kernel_decorator(
    jax.ShapeDtypeStruct((_NC, _N, _HID), jnp.float32),
    [
        pltpu.VMEM((_K, _CHUNK), jnp.int32),
        pltpu.VMEM((_K, _CHUNK), jnp.int32),
    ] + [pltpu.VMEM((_CHUNK, _HID), jnp.float32)] * _NBUF
      + [pltpu.SemaphoreType.DMA] * (2 * _NBUF)
      + [pltpu.VMEM_SHARED((_N, _HID), jnp.float32)] * 2,
)
def _sc_scatter_rows(table_hbm, src_hbm, dst_hbm, zeros_hbm, out_hbm,
                     srcv, dstv, *rest):
    rows = rest[:_NBUF]
    gsem = rest[_NBUF:2 * _NBUF]
    ssem = rest[2 * _NBUF:3 * _NBUF]
    acc = rest[3 * _NBUF]
    tab = rest[3 * _NBUF + 1]
    c = lax.axis_index("c")
    s = lax.axis_index("s")
    wid = c * _NS + s
    r0 = s * _RPT

    pltpu.sync_copy(zeros_hbm.at[pl.ds(r0, _RPT)], acc.at[pl.ds(r0, _RPT)])
    pltpu.sync_copy(table_hbm.at[pl.ds(r0, _RPT)], tab.at[pl.ds(r0, _RPT)])
    pltpu.sync_copy(src_hbm.at[wid], srcv)
    pltpu.sync_copy(dst_hbm.at[wid], dstv)
    plsc.subcore_barrier()

    for b in range(_NBUF):
        pltpu.async_copy(tab.at[srcv.at[b]], rows[b], gsem[b])

    def _step(it, _):
        for b in range(_NBUF):
            j = it * _NBUF + b
            pltpu.make_async_copy(tab.at[srcv.at[j]], rows[b],
                                  gsem[b]).wait()
            pltpu.async_copy(rows[b], acc.at[dstv.at[j]], ssem[b], add=True)
        for b in range(_NBUF):
            j = it * _NBUF + b
            pltpu.make_async_copy(rows[b], acc.at[dstv.at[j]], ssem[b]).wait()
            pltpu.async_copy(tab.at[srcv.at[j + _NBUF]], rows[b], gsem[b])
        return 0

    lax.fori_loop(0, _K // _NBUF - 1, _step, 0)
    for b in range(_NBUF):
        j = _K - _NBUF + b
        pltpu.make_async_copy(tab.at[srcv.at[j]], rows[b], gsem[b]).wait()
        pltpu.async_copy(rows[b], acc.at[dstv.at[j]], ssem[b], add=True)
    for b in range(_NBUF):
        j = _K - _NBUF + b
        pltpu.make_async_copy(rows[b], acc.at[dstv.at[j]], ssem[b]).wait()

    plsc.subcore_barrier()
    pltpu.sync_copy(acc.at[pl.ds(r0, _RPT)], out_hbm.at[c, pl.ds(r0, _RPT)])


def _tc_prepare1_body(deg_parts, x, w1, table1, dinv):
    deg = 1.0 + jnp.sum(deg_parts[...], axis=0)
    di = lax.rsqrt(deg)
    h = jnp.dot(x[...], w1[...], preferred_element_type=jnp.float32)
    table1[...] = h * di[:, None]
    dinv[...] = di[:, None]


def _tc_prepare2_body(accs, table1, dinv, b1, wcat, table2):
    di = dinv[...]
    pre = (accs[0] + accs[1] + table1[...]) * di + b1[...][None, :]
    h = jnp.maximum(pre, 0.0)
    table2[...] = jnp.dot(h, wcat[...], preferred_element_type=jnp.float32) * di


def _tc_final_body(accs, table2, dinv, bcat, mu, lv):
    res = (accs[0] + accs[1] + table2[...]) * dinv[...] + bcat[...][None, :]
    mu[...] = res[:, :_LAT]
    lv[...] = res[:, _LAT:]


def kernel(x, edge_index, W1, b1, W_mu, b_mu, W_lv, b_lv):
    src_w = edge_index[0].reshape(_NW, _K, _CHUNK)
    dst_w = edge_index[1].reshape(_NW, _K, _CHUNK)
    dst_flat = edge_index[1].reshape(_NW, _EPW)

    wcat = jnp.concatenate([W_mu, W_lv], axis=1)
    bcat = jnp.concatenate([b_mu, b_lv])
    zeros_rows = jnp.zeros((_N, _HID), jnp.float32)

    deg_parts = _sc_degree(dst_flat)

    table1, dinv = pl.pallas_call(
        _tc_prepare1_body,
        out_shape=(
            jax.ShapeDtypeStruct((_N, _HID), jnp.float32),
            jax.ShapeDtypeStruct((_N, 1), jnp.float32),
        ),
    )(deg_parts, x, W1)

    acc1 = _sc_scatter_rows(table1, src_w, dst_w, zeros_rows)

    table2 = pl.pallas_call(
        _tc_prepare2_body,
        out_shape=jax.ShapeDtypeStruct((_N, _HID), jnp.float32),
    )(acc1, table1, dinv, b1, wcat)

    acc2 = _sc_scatter_rows(table2, src_w, dst_w, zeros_rows)

    mu, lv = pl.pallas_call(
        _tc_final_body,
        out_shape=(
            jax.ShapeDtypeStruct((_N, _LAT), jnp.float32),
            jax.ShapeDtypeStruct((_N, _LAT), jnp.float32),
        ),
    )(acc2, table2, dinv, bcat)
    return (mu, lv)


# speedup vs baseline: 1.8433x; 1.0307x over previous
import jax
import jax.numpy as jnp
from jax import lax
from jax.experimental import pallas as pl
from jax.experimental.pallas import tpu as pltpu
from jax.experimental.pallas import tpu_sc as plsc

_N = 10000
_E = 320000
_IN_CH = 128
_HID = 32
_LAT = 16

_NC = 2
_NS = 16
_NW = _NC * _NS
_EPW = _E // _NW
_CHUNK = 200
_K = _EPW // _CHUNK
_NBUF = 10
_RPT = _N // _NS

_mesh = plsc.VectorSubcoreMesh(core_axis_name="c", subcore_axis_name="s")
_sc_params = pltpu.CompilerParams(needs_layout_passes=False,
                                  use_tc_tiling_on_sc=False)


def _kernel_decorator(out_type, scratch_types):
    def deco(f):
        return pl.kernel(f, out_type=out_type, mesh=_mesh,
                         scratch_types=scratch_types,
                         compiler_params=_sc_params)
    return deco


@_kernel_decorator(
    jax.ShapeDtypeStruct((_NW, _N), jnp.float32),
    [
        pltpu.VMEM((_EPW,), jnp.int32),
        pltpu.VMEM((_N,), jnp.float32),
    ],
)
def _sc_degree(dst_flat_hbm, out_hbm, dstv, deg):
    wid = lax.axis_index("c") * _NS + lax.axis_index("s")

    def _zero(i, _):
        deg[pl.ds(i * 16, 16)] = jnp.zeros((16,), jnp.float32)
        return 0

    lax.fori_loop(0, _N // 16, _zero, 0)
    pltpu.sync_copy(dst_flat_hbm.at[wid], dstv)
    ones = jnp.ones((16,), jnp.float32)

    def _acc(v, _):
        idx = dstv[pl.ds(v * 16, 16)]
        plsc.addupdate_scatter(deg, [idx], ones)
        return 0

    lax.fori_loop(0, _EPW // 16, _acc, 0)
    pltpu.sync_copy(deg, out_hbm.at[wid])


@_kernel_decorator(
    jax.ShapeDtypeStruct((_NC, _N, _HID), jnp.float32),
    [
        pltpu.VMEM((_K, _CHUNK), jnp.int32),
        pltpu.VMEM((_K, _CHUNK), jnp.int32),
    ] + [pltpu.VMEM((_CHUNK, _HID), jnp.float32)] * _NBUF
      + [pltpu.SemaphoreType.DMA] * (2 * _NBUF)
      + [pltpu.VMEM_SHARED((_N, _HID), jnp.float32)] * 2,
)
def _sc_scatter_rows(table_hbm, src_hbm, dst_hbm, zeros_hbm, out_hbm,
                     srcv, dstv, *rest):
    rows = rest[:_NBUF]
    gsem = rest[_NBUF:2 * _NBUF]
    ssem = rest[2 * _NBUF:3 * _NBUF]
    acc = rest[3 * _NBUF]
    tab = rest[3 * _NBUF + 1]
    c = lax.axis_index("c")
    s = lax.axis_index("s")
    wid = c * _NS + s
    r0 = s * _RPT

    pltpu.sync_copy(zeros_hbm.at[pl.ds(r0, _RPT)], acc.at[pl.ds(r0, _RPT)])
    pltpu.sync_copy(table_hbm.at[pl.ds(r0, _RPT)], tab.at[pl.ds(r0, _RPT)])
    pltpu.sync_copy(src_hbm.at[wid], srcv)
    pltpu.sync_copy(dst_hbm.at[wid], dstv)
    plsc.subcore_barrier()

    for b in range(_NBUF):
        pltpu.async_copy(tab.at[srcv.at[b]], rows[b], gsem[b])

    def _step(it, _):
        for b in range(_NBUF):
            j = it * _NBUF + b
            pltpu.make_async_copy(tab.at[srcv.at[j]], rows[b],
                                  gsem[b]).wait()
            pltpu.async_copy(rows[b], acc.at[dstv.at[j]], ssem[b], add=True)
        for b in range(_NBUF):
            j = it * _NBUF + b
            pltpu.make_async_copy(rows[b], acc.at[dstv.at[j]], ssem[b]).wait()
            pltpu.async_copy(tab.at[srcv.at[j + _NBUF]], rows[b], gsem[b])
        return 0

    lax.fori_loop(0, _K // _NBUF - 1, _step, 0)
    for b in range(_NBUF):
        j = _K - _NBUF + b
        pltpu.make_async_copy(tab.at[srcv.at[j]], rows[b], gsem[b]).wait()
        pltpu.async_copy(rows[b], acc.at[dstv.at[j]], ssem[b], add=True)
    for b in range(_NBUF):
        j = _K - _NBUF + b
        pltpu.make_async_copy(rows[b], acc.at[dstv.at[j]], ssem[b]).wait()

    plsc.subcore_barrier()
    pltpu.sync_copy(acc.at[pl.ds(r0, _RPT)], out_hbm.at[c, pl.ds(r0, _RPT)])


def _tc_prepare1_body(deg_parts, x, w1, table1, dinv):
    deg = 1.0 + jnp.sum(deg_parts[...], axis=0)
    di = lax.rsqrt(deg)
    h = jnp.dot(x[...], w1[...], preferred_element_type=jnp.float32)
    table1[...] = h * di[:, None]
    dinv[...] = di[:, None]


def _tc_prepare2_body(accs, table1, dinv, b1, wcat, table2):
    di = dinv[...]
    pre = (accs[0] + accs[1] + table1[...]) * di + b1[...][None, :]
    h = jnp.maximum(pre, 0.0)
    table2[...] = jnp.dot(h, wcat[...], preferred_element_type=jnp.float32) * di


def _tc_final_body(accs, table2, dinv, bcat, mu, lv):
    res = (accs[0] + accs[1] + table2[...]) * dinv[...] + bcat[...][None, :]
    mu[...] = res[:, :_LAT]
    lv[...] = res[:, _LAT:]


def kernel(x, edge_index, W1, b1, W_mu, b_mu, W_lv, b_lv):
    src_w = edge_index[0].reshape(_NW, _K, _CHUNK)
    dst_w = edge_index[1].reshape(_NW, _K, _CHUNK)
    dst_flat = edge_index[1].reshape(_NW, _EPW)

    wcat = jnp.concatenate([W_mu, W_lv], axis=1)
    bcat = jnp.concatenate([b_mu, b_lv])
    zeros_rows = jnp.zeros((_N, _HID), jnp.float32)

    deg_parts = _sc_degree(dst_flat)

    table1, dinv = pl.pallas_call(
        _tc_prepare1_body,
        out_shape=(
            jax.ShapeDtypeStruct((_N, _HID), jnp.float32),
            jax.ShapeDtypeStruct((_N, 1), jnp.float32),
        ),
    )(deg_parts, x, W1)

    acc1 = _sc_scatter_rows(table1, src_w, dst_w, zeros_rows)

    table2 = pl.pallas_call(
        _tc_prepare2_body,
        out_shape=jax.ShapeDtypeStruct((_N, _HID), jnp.float32),
    )(acc1, table1, dinv, b1, wcat)

    acc2 = _sc_scatter_rows(table2, src_w, dst_w, zeros_rows)

    mu, lv = pl.pallas_call(
        _tc_final_body,
        out_shape=(
            jax.ShapeDtypeStruct((_N, _LAT), jnp.float32),
            jax.ShapeDtypeStruct((_N, _LAT), jnp.float32),
        ),
    )(acc2, table2, dinv, bcat)
    return (mu, lv)


# speedup vs baseline: 1.9214x; 1.0423x over previous
import jax
import jax.numpy as jnp
from jax import lax
from jax.experimental import pallas as pl
from jax.experimental.pallas import tpu as pltpu
from jax.experimental.pallas import tpu_sc as plsc

_N = 10000
_E = 320000
_IN_CH = 128
_HID = 32
_LAT = 16

_NC = 2
_NS = 16
_NW = _NC * _NS
_EPW = _E // _NW
_CHUNK = 200
_K = _EPW // _CHUNK
_NBUF = 10
_RPT = _N // _NS

_mesh = plsc.VectorSubcoreMesh(core_axis_name="c", subcore_axis_name="s")
_sc_params = pltpu.CompilerParams(needs_layout_passes=False,
                                  use_tc_tiling_on_sc=False)


def _kernel_decorator(out_type, scratch_types):
    def deco(f):
        return pl.kernel(f, out_type=out_type, mesh=_mesh,
                         scratch_types=scratch_types,
                         compiler_params=_sc_params)
    return deco


@_kernel_decorator(
    jax.ShapeDtypeStruct((_NW, _N), jnp.float32),
    [
        pltpu.VMEM((_EPW,), jnp.int32),
        pltpu.VMEM((_N,), jnp.float32),
    ],
)
def _sc_degree(dst_flat_hbm, out_hbm, dstv, deg):
    wid = lax.axis_index("c") * _NS + lax.axis_index("s")

    def _zero(i, _):
        deg[pl.ds(i * 16, 16)] = jnp.zeros((16,), jnp.float32)
        return 0

    lax.fori_loop(0, _N // 16, _zero, 0)
    pltpu.sync_copy(dst_flat_hbm.at[wid], dstv)
    ones = jnp.ones((16,), jnp.float32)

    def _acc(v, _):
        idx = dstv[pl.ds(v * 16, 16)]
        plsc.addupdate_scatter(deg, [idx], ones)
        return 0

    lax.fori_loop(0, _EPW // 16, _acc, 0)
    pltpu.sync_copy(deg, out_hbm.at[wid])


@_kernel_decorator(
    jax.ShapeDtypeStruct((_NC, _N, _HID), jnp.float32),
    [
        pltpu.VMEM((_K, _CHUNK), jnp.int32),
        pltpu.VMEM((_K, _CHUNK), jnp.int32),
    ] + [pltpu.VMEM((_CHUNK, _HID), jnp.float32)] * _NBUF
      + [pltpu.SemaphoreType.DMA] * (2 * _NBUF)
      + [pltpu.VMEM_SHARED((_N, _HID), jnp.float32)] * 2,
)
def _sc_scatter_rows(table_hbm, src_hbm, dst_hbm, zeros_hbm, out_hbm,
                     srcv, dstv, *rest):
    rows = rest[:_NBUF]
    gsem = rest[_NBUF:2 * _NBUF]
    ssem = rest[2 * _NBUF:3 * _NBUF]
    acc = rest[3 * _NBUF]
    tab = rest[3 * _NBUF + 1]
    c = lax.axis_index("c")
    s = lax.axis_index("s")
    wid = c * _NS + s
    r0 = s * _RPT

    pltpu.sync_copy(zeros_hbm.at[pl.ds(r0, _RPT)], acc.at[pl.ds(r0, _RPT)])
    pltpu.sync_copy(table_hbm.at[pl.ds(r0, _RPT)], tab.at[pl.ds(r0, _RPT)])
    pltpu.sync_copy(src_hbm.at[wid], srcv)
    pltpu.sync_copy(dst_hbm.at[wid], dstv)
    plsc.subcore_barrier()

    def _src(b):
        return table_hbm if b < 2 else tab

    for b in range(_NBUF):
        pltpu.async_copy(_src(b).at[srcv.at[b]], rows[b], gsem[b])

    def _step(it, _):
        for b in range(_NBUF):
            j = it * _NBUF + b
            pltpu.make_async_copy(_src(b).at[srcv.at[j]], rows[b],
                                  gsem[b]).wait()
            pltpu.async_copy(rows[b], acc.at[dstv.at[j]], ssem[b], add=True)
        for b in range(_NBUF):
            j = it * _NBUF + b
            pltpu.make_async_copy(rows[b], acc.at[dstv.at[j]], ssem[b]).wait()
            pltpu.async_copy(_src(b).at[srcv.at[j + _NBUF]], rows[b], gsem[b])
        return 0

    lax.fori_loop(0, _K // _NBUF - 1, _step, 0)
    for b in range(_NBUF):
        j = _K - _NBUF + b
        pltpu.make_async_copy(_src(b).at[srcv.at[j]], rows[b], gsem[b]).wait()
        pltpu.async_copy(rows[b], acc.at[dstv.at[j]], ssem[b], add=True)
    for b in range(_NBUF):
        j = _K - _NBUF + b
        pltpu.make_async_copy(rows[b], acc.at[dstv.at[j]], ssem[b]).wait()

    plsc.subcore_barrier()
    pltpu.sync_copy(acc.at[pl.ds(r0, _RPT)], out_hbm.at[c, pl.ds(r0, _RPT)])


def _tc_prepare1_body(deg_parts, x, w1, table1, dinv):
    deg = 1.0 + jnp.sum(deg_parts[...], axis=0)
    di = lax.rsqrt(deg)
    h = jnp.dot(x[...], w1[...], preferred_element_type=jnp.float32)
    table1[...] = h * di[:, None]
    dinv[...] = di[:, None]


def _tc_prepare2_body(accs, table1, dinv, b1, wcat, table2):
    di = dinv[...]
    pre = (accs[0] + accs[1] + table1[...]) * di + b1[...][None, :]
    h = jnp.maximum(pre, 0.0)
    table2[...] = jnp.dot(h, wcat[...], preferred_element_type=jnp.float32) * di


def _tc_final_body(accs, table2, dinv, bcat, mu, lv):
    res = (accs[0] + accs[1] + table2[...]) * dinv[...] + bcat[...][None, :]
    mu[...] = res[:, :_LAT]
    lv[...] = res[:, _LAT:]


def kernel(x, edge_index, W1, b1, W_mu, b_mu, W_lv, b_lv):
    src_w = edge_index[0].reshape(_NW, _K, _CHUNK)
    dst_w = edge_index[1].reshape(_NW, _K, _CHUNK)
    dst_flat = edge_index[1].reshape(_NW, _EPW)

    wcat = jnp.concatenate([W_mu, W_lv], axis=1)
    bcat = jnp.concatenate([b_mu, b_lv])
    zeros_rows = jnp.zeros((_N, _HID), jnp.float32)

    deg_parts = _sc_degree(dst_flat)

    table1, dinv = pl.pallas_call(
        _tc_prepare1_body,
        out_shape=(
            jax.ShapeDtypeStruct((_N, _HID), jnp.float32),
            jax.ShapeDtypeStruct((_N, 1), jnp.float32),
        ),
    )(deg_parts, x, W1)

    acc1 = _sc_scatter_rows(table1, src_w, dst_w, zeros_rows)

    table2 = pl.pallas_call(
        _tc_prepare2_body,
        out_shape=jax.ShapeDtypeStruct((_N, _HID), jnp.float32),
    )(acc1, table1, dinv, b1, wcat)

    acc2 = _sc_scatter_rows(table2, src_w, dst_w, zeros_rows)

    mu, lv = pl.pallas_call(
        _tc_final_body,
        out_shape=(
            jax.ShapeDtypeStruct((_N, _LAT), jnp.float32),
            jax.ShapeDtypeStruct((_N, _LAT), jnp.float32),
        ),
    )(acc2, table2, dinv, bcat)
    return (mu, lv)
